# baseline (speedup 1.0000x reference)
import functools

import jax
import jax.numpy as jnp
from jax import lax
from jax.experimental import pallas as pl
from jax.experimental.pallas import tpu as pltpu
from jax.experimental.pallas import tpu_sc as plsc

D_VOCAB = 100000
D_MODEL = 1024


@functools.cache
def _make_gather(V, D, B):
    info = plsc.get_sparse_core_info()
    NC, NS = info.num_cores, info.num_subcores
    NW = NC * NS
    assert B % NW == 0
    b_per_w = B // NW
    CHUNK = 16
    NBUF = 4
    n_chunks = b_per_w // CHUNK
    assert n_chunks % NBUF == 0 and n_chunks >= 2 * NBUF

    mesh = plsc.VectorSubcoreMesh(core_axis_name="c", subcore_axis_name="s")

    @functools.partial(
        pl.kernel,
        mesh=mesh,
        out_type=jax.ShapeDtypeStruct((B, D), jnp.float32),
        scratch_types=[
            pltpu.VMEM((b_per_w,), jnp.int32),
            pltpu.VMEM((NBUF, CHUNK, D), jnp.float32),
            pltpu.SemaphoreType.DMA,
            pltpu.SemaphoreType.DMA,
            pltpu.SemaphoreType.DMA,
            pltpu.SemaphoreType.DMA,
            pltpu.SemaphoreType.DMA,
            pltpu.SemaphoreType.DMA,
            pltpu.SemaphoreType.DMA,
            pltpu.SemaphoreType.DMA,
        ],
    )
    def k(idx_hbm, table_hbm, out_hbm, idx_v, rows_v,
          g0, g1, g2, g3, o0, o1, o2, o3):
        gsem = (g0, g1, g2, g3)
        osem = (o0, o1, o2, o3)
        wid = lax.axis_index("s") * NC + lax.axis_index("c")
        base = pl.multiple_of(wid * b_per_w, b_per_w)
        w_per_row = idx_hbm.shape[1] // b_per_w
        row = wid // w_per_row
        col = pl.multiple_of((wid % w_per_row) * b_per_w, b_per_w)
        pltpu.sync_copy(idx_hbm.at[row, pl.ds(col, b_per_w)], idx_v)

        def issue_gather(ci, b):
            off = pl.multiple_of(ci * CHUNK, CHUNK)
            pltpu.async_copy(
                table_hbm.at[idx_v.at[pl.ds(off, CHUNK)]], rows_v.at[b], gsem[b]
            )

        def wait_gather(b):
            pltpu.make_async_copy(
                table_hbm.at[idx_v.at[pl.ds(0, CHUNK)]], rows_v.at[b], gsem[b]
            ).wait()

        def issue_store(ci, b):
            off = pl.multiple_of(ci * CHUNK, CHUNK)
            pltpu.async_copy(
                rows_v.at[b], out_hbm.at[pl.ds(base + off, CHUNK)], osem[b]
            )

        def wait_store(b):
            pltpu.make_async_copy(
                rows_v.at[b], out_hbm.at[pl.ds(base, CHUNK)], osem[b]
            ).wait()

        sid = lax.axis_index("s")
        pair = wid // 2
        wbase = pl.multiple_of(pair * (2 * b_per_w), b_per_w)

        @pl.when(sid % 2 == 0)
        def _reader():
            def rbody(q, carry):
                for j in range(4):
                    qq = q * 4 + j
                    off = pl.multiple_of(lax.rem(qq, 32) * CHUNK, CHUNK)
                    pltpu.async_copy(
                        table_hbm.at[idx_v.at[pl.ds(off, CHUNK)]],
                        rows_v.at[j], gsem[j],
                    )
                return carry
            lax.fori_loop(0, 16, rbody, jnp.int32(0))
            def rdrain(q, carry):
                for j in range(4):
                    pltpu.make_async_copy(
                        table_hbm.at[idx_v.at[pl.ds(0, CHUNK)]],
                        rows_v.at[j], gsem[j],
                    ).wait()
                return carry
            lax.fori_loop(0, 16, rdrain, jnp.int32(0))

        @pl.when(sid % 2 == 1)
        def _writer():
            def wbody(q, carry):
                for j in range(4):
                    qq = q * 4 + j
                    off = pl.multiple_of(lax.rem(qq, 64) * CHUNK, CHUNK)
                    pltpu.async_copy(
                        rows_v.at[j], out_hbm.at[pl.ds(wbase + off, CHUNK)],
                        osem[j],
                    )
                return carry
            lax.fori_loop(0, 16, wbody, jnp.int32(0))
            def wdrain(q, carry):
                for j in range(4):
                    pltpu.make_async_copy(
                        rows_v.at[j], out_hbm.at[pl.ds(wbase, CHUNK)], osem[j],
                    ).wait()
                return carry
            lax.fori_loop(0, 16, wdrain, jnp.int32(0))

    return k


def kernel(x, W_E):
    B, S = x.shape
    idx = x.astype(jnp.int32)
    out = _make_gather(W_E.shape[0], W_E.shape[1], B * S)(idx, W_E)
    return out.reshape(B, S, W_E.shape[1])


# speedup vs baseline: 1.0034x; 1.0034x over previous
import functools

import jax
import jax.numpy as jnp
from jax import lax
from jax.experimental import pallas as pl
from jax.experimental.pallas import tpu as pltpu
from jax.experimental.pallas import tpu_sc as plsc

D_VOCAB = 100000
D_MODEL = 1024


@functools.cache
def _make_gather(V, D, B):
    info = plsc.get_sparse_core_info()
    NC, NS = info.num_cores, info.num_subcores
    NW = NC * NS
    assert B % NW == 0
    b_per_w = B // NW
    CHUNK = 16
    NBUF = 4
    n_chunks = b_per_w // CHUNK
    assert n_chunks % NBUF == 0 and n_chunks >= 2 * NBUF

    mesh = plsc.VectorSubcoreMesh(core_axis_name="c", subcore_axis_name="s")

    @functools.partial(
        pl.kernel,
        mesh=mesh,
        out_type=jax.ShapeDtypeStruct((B, D), jnp.float32),
        scratch_types=[
            pltpu.VMEM((b_per_w,), jnp.int32),
            pltpu.VMEM((NBUF, CHUNK, D), jnp.float32),
            pltpu.SemaphoreType.DMA,
            pltpu.SemaphoreType.DMA,
            pltpu.SemaphoreType.DMA,
            pltpu.SemaphoreType.DMA,
            pltpu.SemaphoreType.DMA,
            pltpu.SemaphoreType.DMA,
            pltpu.SemaphoreType.DMA,
            pltpu.SemaphoreType.DMA,
        ],
    )
    def k(idx_hbm, table_hbm, out_hbm, idx_v, rows_v,
          g0, g1, g2, g3, o0, o1, o2, o3):
        gsem = (g0, g1, g2, g3)
        osem = (o0, o1, o2, o3)
        wid = lax.axis_index("s") * NC + lax.axis_index("c")
        base = pl.multiple_of(wid * b_per_w, b_per_w)
        w_per_row = idx_hbm.shape[1] // b_per_w
        row = wid // w_per_row
        col = pl.multiple_of((wid % w_per_row) * b_per_w, b_per_w)
        pltpu.sync_copy(idx_hbm.at[row, pl.ds(col, b_per_w)], idx_v)

        def issue_gather(ci, b):
            off = pl.multiple_of(ci * CHUNK, CHUNK)
            pltpu.async_copy(
                table_hbm.at[idx_v.at[pl.ds(off, CHUNK)]], rows_v.at[b], gsem[b]
            )

        def wait_gather(ci, b):
            off = pl.multiple_of(ci * CHUNK, CHUNK)
            pltpu.make_async_copy(
                table_hbm.at[idx_v.at[pl.ds(off, CHUNK)]], rows_v.at[b], gsem[b]
            ).wait()

        def issue_store(ci, b):
            off = pl.multiple_of(ci * CHUNK, CHUNK)
            pltpu.async_copy(
                rows_v.at[b], out_hbm.at[pl.ds(base + off, CHUNK)], osem[b]
            )

        def wait_store(ci, b):
            off = pl.multiple_of(ci * CHUNK, CHUNK)
            pltpu.make_async_copy(
                rows_v.at[b], out_hbm.at[pl.ds(base + off, CHUNK)], osem[b]
            ).wait()

        for ci in (0, 1, 2):
            issue_gather(jnp.int32(ci), ci)
        wait_gather(jnp.int32(0), 0)
        issue_store(jnp.int32(0), 0)
        issue_gather(jnp.int32(3), 3)

        def body(i, carry):
            ci0 = 1 + i * NBUF
            for j in range(NBUF):
                ci = ci0 + j
                b = (1 + j) % NBUF
                bn = j % NBUF
                wait_gather(ci, b)
                issue_store(ci, b)
                wait_store(ci - 1, bn)
                issue_gather(ci + 3, bn)
            return carry

        lax.fori_loop(0, (n_chunks - 4) // NBUF, body, jnp.int32(0))

        for ci in (n_chunks - 3, n_chunks - 2, n_chunks - 1):
            b = ci % NBUF
            wait_gather(jnp.int32(ci), b)
            issue_store(jnp.int32(ci), b)
        for q in range(NBUF):
            ci = n_chunks - NBUF + q
            wait_store(jnp.int32(ci), ci % NBUF)

    return k


def kernel(x, W_E):
    B, S = x.shape
    idx = x.astype(jnp.int32)
    out = _make_gather(W_E.shape[0], W_E.shape[1], B * S)(idx, W_E)
    return out.reshape(B, S, W_E.shape[1])
